# baseline (speedup 1.0000x reference)
import functools

import jax
import jax.numpy as jnp
from jax import lax
from jax.experimental import pallas as pl
from jax.experimental.pallas import tpu as pltpu
from jax.experimental.pallas import tpu_sc as plsc

N = 10000
E = 320000
IN_F = 128
HID = 256
OUT_F = 121

NC = 2
NS = 16
NW = NC * NS
CHUNK = 128
EP = 323584
QA = 53
QB = 105
RPT = N // NS
NA = N + 8

BF = jnp.bfloat16


def _mesh():
    return plsc.VectorSubcoreMesh(core_axis_name="c", subcore_axis_name="s",
                                  num_cores=NC, num_subcores=NS)


def _pipeline(table, mixed_w, ib0, ib1, rb0, rb1, si0, si1, sg0, sg1,
              acc, nchunk):
    ibufs, rbufs = (ib0, ib1), (rb0, rb1)
    isems, gsems = (si0, si1), (sg0, sg1)

    def iload(c, p):
        pltpu.async_copy(mixed_w.at[c], ibufs[p], isems[p])

    def iwait(c, p):
        pltpu.make_async_copy(mixed_w.at[c], ibufs[p], isems[p]).wait()

    def gstart(p):
        pltpu.async_copy(table.at[ibufs[p].at[0]], rbufs[p], gsems[p])

    def gwait(p):
        pltpu.make_async_copy(table.at[ibufs[p].at[0]], rbufs[p],
                              gsems[p]).wait()

    def scat(p):
        pltpu.sync_copy(rbufs[p], acc.at[ibufs[p].at[1]], add=True)

    iload(0, 0)
    iwait(0, 0)
    gstart(0)
    iload(1, 1)

    def body(c, carry):
        def stage(p):
            iwait(c, p)
            gstart(p)
            gwait(1 - p)
            scat(1 - p)

            @pl.when(c < nchunk - 1)
            def _():
                iload(c + 1, 1 - p)

        @pl.when(c % 2 == 1)
        def _():
            stage(1)

        @pl.when(c % 2 == 0)
        def _():
            stage(0)

        return carry

    lax.fori_loop(1, nchunk, body, 0)
    p = (nchunk - 1) % 2
    gwait(p)
    scat(p)


@functools.lru_cache(maxsize=None)
def _make_agg(D):
    @functools.partial(
        pl.kernel,
        out_type=jax.ShapeDtypeStruct((2 * N, D), BF),
        mesh=_mesh(),
        scratch_types=[
            pltpu.VMEM((2, CHUNK), jnp.int32),
            pltpu.VMEM((2, CHUNK), jnp.int32),
            pltpu.VMEM((CHUNK, D), BF),
            pltpu.VMEM((CHUNK, D), BF),
            pltpu.VMEM_SHARED((NA, D), BF),
            pltpu.SemaphoreType.DMA,
            pltpu.SemaphoreType.DMA,
            pltpu.SemaphoreType.DMA,
            pltpu.SemaphoreType.DMA,
            pltpu.SemaphoreType.DMA,
        ],
        compiler_params=pltpu.CompilerParams(use_tc_tiling_on_sc=False),
    )
    def agg(table, mixed_a, mixed_b, zeros, out,
            ib0, ib1, rb0, rb1, acc, si0, si1, sg0, sg1, semz):
        cid = lax.axis_index("c")
        sid = lax.axis_index("s")
        zcp = pltpu.async_copy(zeros.at[pl.ds(sid * RPT, RPT)],
                               acc.at[pl.ds(sid * RPT, RPT)], semz)
        zcp.wait()
        plsc.subcore_barrier()

        @pl.when(cid == 0)
        def _():
            _pipeline(table, mixed_a.at[sid], ib0, ib1, rb0, rb1,
                      si0, si1, sg0, sg1, acc, QA)

        @pl.when(cid == 1)
        def _():
            _pipeline(table, mixed_b.at[sid], ib0, ib1, rb0, rb1,
                      si0, si1, sg0, sg1, acc, QB)

        plsc.subcore_barrier()
        pltpu.sync_copy(acc.at[pl.ds(sid * RPT, RPT)],
                        out.at[pl.ds(cid * N + sid * RPT, RPT)])

    return agg


BM = 2000
GRID = N // BM


def _l1_body(pa, pb, x, wl, wr, b, hl_ref, hr_ref, hb_ref, inv_ref):
    s = pa[...].astype(jnp.float32) + pb[...].astype(jnp.float32)
    cnt = s[:, IN_F:IN_F + 1]
    inv = 1.0 / jnp.maximum(cnt, 1.0)
    agg = s[:, :IN_F] * inv
    h = (jnp.dot(agg, wl[...], preferred_element_type=jnp.float32)
         + jnp.dot(x[...], wr[...], preferred_element_type=jnp.float32)
         + b[...])
    h = jnp.maximum(h, 0.0)
    hl_ref[...] = h[:, :128]
    hr_ref[...] = h[:, 128:]
    hb_ref[...] = h.astype(BF)
    inv_ref[...] = inv


def _tc_l1(parts1, x, Wl1, Wr1, b1r):
    return pl.pallas_call(
        _l1_body,
        grid=(GRID,),
        in_specs=[
            pl.BlockSpec((BM, 160), lambda i: (i, 0)),
            pl.BlockSpec((BM, 160), lambda i: (i + GRID, 0)),
            pl.BlockSpec((BM, IN_F), lambda i: (i, 0)),
            pl.BlockSpec((IN_F, HID), lambda i: (0, 0)),
            pl.BlockSpec((IN_F, HID), lambda i: (0, 0)),
            pl.BlockSpec((1, HID), lambda i: (0, 0)),
        ],
        out_specs=[
            pl.BlockSpec((BM, 128), lambda i: (i, 0)),
            pl.BlockSpec((BM, 128), lambda i: (i, 0)),
            pl.BlockSpec((BM, HID), lambda i: (i, 0)),
            pl.BlockSpec((BM, 1), lambda i: (i, 0)),
        ],
        out_shape=[
            jax.ShapeDtypeStruct((N, 128), jnp.float32),
            jax.ShapeDtypeStruct((N, 128), jnp.float32),
            jax.ShapeDtypeStruct((N, HID), BF),
            jax.ShapeDtypeStruct((N, 1), jnp.float32),
        ],
    )(parts1, parts1, x, Wl1, Wr1, b1r)


def _l2_body(pa, pb, inv, h1l, h1r,
             w2, wr2a, wr2b, b2, wl3p, wr3p, b3p, z_ref, r_ref):
    iv = inv[...]
    agg = (pa[...].astype(jnp.float32) + pb[...].astype(jnp.float32)) * iv
    h2 = (jnp.dot(agg, w2[...], preferred_element_type=jnp.float32)
          + jnp.dot(h1l[...], wr2a[...], preferred_element_type=jnp.float32)
          + jnp.dot(h1r[...], wr2b[...], preferred_element_type=jnp.float32)
          + b2[...])
    h2 = jnp.maximum(h2, 0.0)
    z_ref[...] = jnp.dot(h2, wl3p[...],
                         preferred_element_type=jnp.float32).astype(BF)
    r_ref[...] = (jnp.dot(h2, wr3p[...], preferred_element_type=jnp.float32)
                  + b3p[...])


def _tc_l2(p2, inv, h1l, h1r, w2, wr2a, wr2b, b2r, wl3p, wr3p, b3pr):
    blk = lambda i: (i, 0)
    blk2 = lambda i: (i + GRID, 0)
    full = lambda i: (0, 0)
    return pl.pallas_call(
        _l2_body,
        grid=(GRID,),
        in_specs=[
            pl.BlockSpec((BM, HID), blk),
            pl.BlockSpec((BM, HID), blk2),
            pl.BlockSpec((BM, 1), blk),
            pl.BlockSpec((BM, 128), blk),
            pl.BlockSpec((BM, 128), blk),
            pl.BlockSpec((HID, HID), full),
            pl.BlockSpec((128, HID), full),
            pl.BlockSpec((128, HID), full),
            pl.BlockSpec((1, HID), full),
            pl.BlockSpec((HID, 128), full),
            pl.BlockSpec((HID, 128), full),
            pl.BlockSpec((1, 128), full),
        ],
        out_specs=[
            pl.BlockSpec((BM, 128), blk),
            pl.BlockSpec((BM, 128), blk),
        ],
        out_shape=[
            jax.ShapeDtypeStruct((N, 128), BF),
            jax.ShapeDtypeStruct((N, 128), jnp.float32),
        ],
    )(p2, p2, inv, h1l, h1r, w2, wr2a, wr2b, b2r, wl3p, wr3p, b3pr)


def _l3_body(qa, qb, inv, r, out_ref):
    q = qa[...].astype(jnp.float32) + qb[...].astype(jnp.float32)
    v = q * inv[...] + r[...]
    out_ref[...] = jnp.maximum(v, 0.0)[:, :OUT_F]


def _tc_l3(parts3, inv, r):
    blk = lambda i: (i, 0)
    return pl.pallas_call(
        _l3_body,
        grid=(GRID,),
        in_specs=[
            pl.BlockSpec((BM, 128), blk),
            pl.BlockSpec((BM, 128), lambda i: (i + GRID, 0)),
            pl.BlockSpec((BM, 1), blk),
            pl.BlockSpec((BM, 128), blk),
        ],
        out_specs=pl.BlockSpec((BM, OUT_F), blk),
        out_shape=jax.ShapeDtypeStruct((N, OUT_F), jnp.float32),
    )(parts3, parts3, inv, r)


def kernel(x, edge_index, Wl1, Wr1, b1, Wl2, Wr2, b2, Wl3, Wr3, b3):
    ei = edge_index.astype(jnp.int32)
    src, dst = ei[0], ei[1]

    pad = EP - E
    src_p = jnp.concatenate([src, jnp.zeros((pad,), jnp.int32)])
    dst_p = jnp.concatenate([dst, jnp.full((pad,), N, jnp.int32)])
    cut = NS * QA * CHUNK
    mixed_a = jnp.stack([src_p[:cut].reshape(NS, QA, CHUNK),
                         dst_p[:cut].reshape(NS, QA, CHUNK)], axis=2)
    mixed_b = jnp.stack([src_p[cut:].reshape(NS, QB, CHUNK),
                         dst_p[cut:].reshape(NS, QB, CHUNK)], axis=2)

    x_pad = jnp.concatenate(
        [x, jnp.ones((N, 1), jnp.float32), jnp.zeros((N, 31), jnp.float32)],
        axis=1).astype(BF)
    z160 = jnp.zeros((N, 160), BF)
    z256 = jnp.zeros((N, HID), BF)
    z128 = jnp.zeros((N, 128), BF)

    b1r = b1.reshape(1, HID)
    wr2a, wr2b = Wr2[:128], Wr2[128:]
    b2r = b2.reshape(1, HID)
    wl3p = jnp.pad(Wl3, ((0, 0), (0, 128 - OUT_F)))
    wr3p = jnp.pad(Wr3, ((0, 0), (0, 128 - OUT_F)))
    b3pr = jnp.pad(b3, (0, 128 - OUT_F)).reshape(1, 128)

    agg160 = _make_agg(160)
    agg256 = _make_agg(HID)
    agg128 = _make_agg(128)

    parts1 = agg160(x_pad, mixed_a, mixed_b, z160)
    h1l, h1r, h1b, inv = _tc_l1(parts1, x, Wl1, Wr1, b1r)

    p2 = agg256(h1b, mixed_a, mixed_b, z256)
    z, r = _tc_l2(p2, inv, h1l, h1r, Wl2, wr2a, wr2b, b2r, wl3p, wr3p, b3pr)

    parts3 = agg128(z, mixed_a, mixed_b, z128)
    return _tc_l3(parts3, inv, r)


# speedup vs baseline: 1.1310x; 1.1310x over previous
import functools

import jax
import jax.numpy as jnp
from jax import lax
from jax.experimental import pallas as pl
from jax.experimental.pallas import tpu as pltpu
from jax.experimental.pallas import tpu_sc as plsc

N = 10000
E = 320000
IN_F = 128
HID = 256
OUT_F = 121

NC = 2
NS = 16
NW = NC * NS
CHUNK = 128
EP = 323584
QA = 105
QB = 53
RPT = N // NS
NA = N + 8

BF = jnp.bfloat16


def _mesh():
    return plsc.VectorSubcoreMesh(core_axis_name="c", subcore_axis_name="s",
                                  num_cores=NC, num_subcores=NS)


def _pipeline(table, mixed_w, ib0, ib1, rb0, rb1, si0, si1, sg0, sg1,
              acc, nchunk):
    ibufs, rbufs = (ib0, ib1), (rb0, rb1)
    isems, gsems = (si0, si1), (sg0, sg1)

    def iload(c, p):
        pltpu.async_copy(mixed_w.at[c], ibufs[p], isems[p])

    def iwait(c, p):
        pltpu.make_async_copy(mixed_w.at[c], ibufs[p], isems[p]).wait()

    def gstart(p):
        pltpu.async_copy(table.at[ibufs[p].at[0]], rbufs[p], gsems[p])

    def gwait(p):
        pltpu.make_async_copy(table.at[ibufs[p].at[0]], rbufs[p],
                              gsems[p]).wait()

    def scat(p):
        pltpu.sync_copy(rbufs[p], acc.at[ibufs[p].at[1]], add=True)

    iload(0, 0)
    iwait(0, 0)
    gstart(0)
    iload(1, 1)

    def body(c, carry):
        def stage(p):
            iwait(c, p)
            gstart(p)
            gwait(1 - p)
            scat(1 - p)

            @pl.when(c < nchunk - 1)
            def _():
                iload(c + 1, 1 - p)

        @pl.when(c % 2 == 1)
        def _():
            stage(1)

        @pl.when(c % 2 == 0)
        def _():
            stage(0)

        return carry

    lax.fori_loop(1, nchunk, body, 0)
    p = (nchunk - 1) % 2
    gwait(p)
    scat(p)


@functools.lru_cache(maxsize=None)
def _make_agg(D):
    @functools.partial(
        pl.kernel,
        out_type=jax.ShapeDtypeStruct((2 * N, D), BF),
        mesh=_mesh(),
        scratch_types=[
            pltpu.VMEM((2, CHUNK), jnp.int32),
            pltpu.VMEM((2, CHUNK), jnp.int32),
            pltpu.VMEM((CHUNK, D), BF),
            pltpu.VMEM((CHUNK, D), BF),
            pltpu.VMEM_SHARED((NA, D), BF),
            pltpu.SemaphoreType.DMA,
            pltpu.SemaphoreType.DMA,
            pltpu.SemaphoreType.DMA,
            pltpu.SemaphoreType.DMA,
            pltpu.SemaphoreType.DMA,
        ],
        compiler_params=pltpu.CompilerParams(use_tc_tiling_on_sc=False),
    )
    def agg(table, mixed_a, mixed_b, zeros, out,
            ib0, ib1, rb0, rb1, acc, si0, si1, sg0, sg1, semz):
        cid = lax.axis_index("c")
        sid = lax.axis_index("s")
        zcp = pltpu.async_copy(zeros.at[pl.ds(sid * RPT, RPT)],
                               acc.at[pl.ds(sid * RPT, RPT)], semz)
        zcp.wait()
        plsc.subcore_barrier()

        @pl.when(cid == 0)
        def _():
            _pipeline(table, mixed_a.at[sid], ib0, ib1, rb0, rb1,
                      si0, si1, sg0, sg1, acc, QA)

        @pl.when(cid == 1)
        def _():
            _pipeline(table, mixed_b.at[sid], ib0, ib1, rb0, rb1,
                      si0, si1, sg0, sg1, acc, QB)

        plsc.subcore_barrier()
        pltpu.sync_copy(acc.at[pl.ds(sid * RPT, RPT)],
                        out.at[pl.ds(cid * N + sid * RPT, RPT)])

    return agg


BM = 2000
GRID = N // BM


def _l1_body(pa, pb, x, wl, wr, b, hl_ref, hr_ref, hb_ref, inv_ref):
    s = pa[...].astype(jnp.float32) + pb[...].astype(jnp.float32)
    cnt = s[:, IN_F:IN_F + 1]
    inv = 1.0 / jnp.maximum(cnt, 1.0)
    agg = s[:, :IN_F] * inv
    h = (jnp.dot(agg, wl[...], preferred_element_type=jnp.float32)
         + jnp.dot(x[...], wr[...], preferred_element_type=jnp.float32)
         + b[...])
    h = jnp.maximum(h, 0.0)
    hl_ref[...] = h[:, :128]
    hr_ref[...] = h[:, 128:]
    hb_ref[...] = h.astype(BF)
    inv_ref[...] = inv


def _tc_l1(parts1, x, Wl1, Wr1, b1r):
    return pl.pallas_call(
        _l1_body,
        grid=(GRID,),
        in_specs=[
            pl.BlockSpec((BM, 160), lambda i: (i, 0)),
            pl.BlockSpec((BM, 160), lambda i: (i + GRID, 0)),
            pl.BlockSpec((BM, IN_F), lambda i: (i, 0)),
            pl.BlockSpec((IN_F, HID), lambda i: (0, 0)),
            pl.BlockSpec((IN_F, HID), lambda i: (0, 0)),
            pl.BlockSpec((1, HID), lambda i: (0, 0)),
        ],
        out_specs=[
            pl.BlockSpec((BM, 128), lambda i: (i, 0)),
            pl.BlockSpec((BM, 128), lambda i: (i, 0)),
            pl.BlockSpec((BM, HID), lambda i: (i, 0)),
            pl.BlockSpec((BM, 1), lambda i: (i, 0)),
        ],
        out_shape=[
            jax.ShapeDtypeStruct((N, 128), jnp.float32),
            jax.ShapeDtypeStruct((N, 128), jnp.float32),
            jax.ShapeDtypeStruct((N, HID), BF),
            jax.ShapeDtypeStruct((N, 1), jnp.float32),
        ],
    )(parts1, parts1, x, Wl1, Wr1, b1r)


def _l2_body(pa, pb, inv, h1l, h1r,
             w2, wr2a, wr2b, b2, wl3p, wr3p, b3p, z_ref, r_ref):
    iv = inv[...]
    agg = (pa[...].astype(jnp.float32) + pb[...].astype(jnp.float32)) * iv
    h2 = (jnp.dot(agg, w2[...], preferred_element_type=jnp.float32)
          + jnp.dot(h1l[...], wr2a[...], preferred_element_type=jnp.float32)
          + jnp.dot(h1r[...], wr2b[...], preferred_element_type=jnp.float32)
          + b2[...])
    h2 = jnp.maximum(h2, 0.0)
    z_ref[...] = jnp.dot(h2, wl3p[...],
                         preferred_element_type=jnp.float32).astype(BF)
    r_ref[...] = (jnp.dot(h2, wr3p[...], preferred_element_type=jnp.float32)
                  + b3p[...])


def _tc_l2(p2, inv, h1l, h1r, w2, wr2a, wr2b, b2r, wl3p, wr3p, b3pr):
    blk = lambda i: (i, 0)
    blk2 = lambda i: (i + GRID, 0)
    full = lambda i: (0, 0)
    return pl.pallas_call(
        _l2_body,
        grid=(GRID,),
        in_specs=[
            pl.BlockSpec((BM, HID), blk),
            pl.BlockSpec((BM, HID), blk2),
            pl.BlockSpec((BM, 1), blk),
            pl.BlockSpec((BM, 128), blk),
            pl.BlockSpec((BM, 128), blk),
            pl.BlockSpec((HID, HID), full),
            pl.BlockSpec((128, HID), full),
            pl.BlockSpec((128, HID), full),
            pl.BlockSpec((1, HID), full),
            pl.BlockSpec((HID, 128), full),
            pl.BlockSpec((HID, 128), full),
            pl.BlockSpec((1, 128), full),
        ],
        out_specs=[
            pl.BlockSpec((BM, 128), blk),
            pl.BlockSpec((BM, 128), blk),
        ],
        out_shape=[
            jax.ShapeDtypeStruct((N, 128), BF),
            jax.ShapeDtypeStruct((N, 128), jnp.float32),
        ],
    )(p2, p2, inv, h1l, h1r, w2, wr2a, wr2b, b2r, wl3p, wr3p, b3pr)


def _l3_body(qa, qb, inv, r, out_ref):
    q = qa[...].astype(jnp.float32) + qb[...].astype(jnp.float32)
    v = q * inv[...] + r[...]
    out_ref[...] = jnp.maximum(v, 0.0)[:, :OUT_F]


def _tc_l3(parts3, inv, r):
    blk = lambda i: (i, 0)
    return pl.pallas_call(
        _l3_body,
        grid=(GRID,),
        in_specs=[
            pl.BlockSpec((BM, 128), blk),
            pl.BlockSpec((BM, 128), lambda i: (i + GRID, 0)),
            pl.BlockSpec((BM, 1), blk),
            pl.BlockSpec((BM, 128), blk),
        ],
        out_specs=pl.BlockSpec((BM, OUT_F), blk),
        out_shape=jax.ShapeDtypeStruct((N, OUT_F), jnp.float32),
    )(parts3, parts3, inv, r)


def kernel(x, edge_index, Wl1, Wr1, b1, Wl2, Wr2, b2, Wl3, Wr3, b3):
    ei = edge_index.astype(jnp.int32)
    src, dst = ei[0], ei[1]

    pad = EP - E
    src_p = jnp.concatenate([src, jnp.zeros((pad,), jnp.int32)])
    dst_p = jnp.concatenate([dst, jnp.full((pad,), N, jnp.int32)])
    cut = NS * QA * CHUNK
    mixed_a = jnp.stack([src_p[:cut].reshape(NS, QA, CHUNK),
                         dst_p[:cut].reshape(NS, QA, CHUNK)], axis=2)
    mixed_b = jnp.stack([src_p[cut:].reshape(NS, QB, CHUNK),
                         dst_p[cut:].reshape(NS, QB, CHUNK)], axis=2)

    x_pad = jnp.concatenate(
        [x, jnp.ones((N, 1), jnp.float32), jnp.zeros((N, 31), jnp.float32)],
        axis=1).astype(BF)
    z160 = jnp.zeros((N, 160), BF)
    z256 = jnp.zeros((N, HID), BF)
    z128 = jnp.zeros((N, 128), BF)

    b1r = b1.reshape(1, HID)
    wr2a, wr2b = Wr2[:128], Wr2[128:]
    b2r = b2.reshape(1, HID)
    wl3p = jnp.pad(Wl3, ((0, 0), (0, 128 - OUT_F)))
    wr3p = jnp.pad(Wr3, ((0, 0), (0, 128 - OUT_F)))
    b3pr = jnp.pad(b3, (0, 128 - OUT_F)).reshape(1, 128)

    agg160 = _make_agg(160)
    agg256 = _make_agg(HID)
    agg128 = _make_agg(128)

    parts1 = agg160(x_pad, mixed_a, mixed_b, z160)
    h1l, h1r, h1b, inv = _tc_l1(parts1, x, Wl1, Wr1, b1r)

    p2 = agg256(h1b, mixed_a, mixed_b, z256)
    z, r = _tc_l2(p2, inv, h1l, h1r, Wl2, wr2a, wr2b, b2r, wl3p, wr3p, b3pr)

    parts3 = agg128(z, mixed_a, mixed_b, z128)
    return _tc_l3(parts3, inv, r)


# speedup vs baseline: 1.1521x; 1.0187x over previous
import functools

import jax
import jax.numpy as jnp
from jax import lax
from jax.experimental import pallas as pl
from jax.experimental.pallas import tpu as pltpu
from jax.experimental.pallas import tpu_sc as plsc

N = 10000
E = 320000
IN_F = 128
HID = 256
OUT_F = 121

NC = 2
NS = 16
NW = NC * NS
CHUNK = 128
EP = 323584
TCH = EP // CHUNK
RPT = N // NS
NA = N + 8

BF = jnp.bfloat16


def _mesh():
    return plsc.VectorSubcoreMesh(core_axis_name="c", subcore_axis_name="s",
                                  num_cores=NC, num_subcores=NS)


def _pipeline(table, mixed_g, base, ib0, ib1, rb0, rb1, si0, si1, sg0, sg1,
              acc, nchunk):
    ibufs, rbufs = (ib0, ib1), (rb0, rb1)
    isems, gsems = (si0, si1), (sg0, sg1)

    def iload(c, p):
        pltpu.async_copy(mixed_g.at[base + c], ibufs[p], isems[p])

    def iwait(c, p):
        pltpu.make_async_copy(mixed_g.at[base + c], ibufs[p],
                              isems[p]).wait()

    def gstart(p):
        pltpu.async_copy(table.at[ibufs[p].at[0]], rbufs[p], gsems[p])

    def gwait(p):
        pltpu.make_async_copy(table.at[ibufs[p].at[0]], rbufs[p],
                              gsems[p]).wait()

    def scat(p):
        pltpu.sync_copy(rbufs[p], acc.at[ibufs[p].at[1]], add=True)

    iload(0, 0)
    iwait(0, 0)
    gstart(0)
    iload(1, 1)

    def body(c, carry):
        def stage(p):
            iwait(c, p)
            gstart(p)
            gwait(1 - p)
            scat(1 - p)

            @pl.when(c < nchunk - 1)
            def _():
                iload(c + 1, 1 - p)

        @pl.when(c % 2 == 1)
        def _():
            stage(1)

        @pl.when(c % 2 == 0)
        def _():
            stage(0)

        return carry

    lax.fori_loop(1, nchunk, body, 0)
    p = (nchunk - 1) % 2
    gwait(p)
    scat(p)


@functools.lru_cache(maxsize=None)
def _make_agg(D, qa, qb):
    @functools.partial(
        pl.kernel,
        out_type=jax.ShapeDtypeStruct((2 * N, D), BF),
        mesh=_mesh(),
        scratch_types=[
            pltpu.VMEM((2, CHUNK), jnp.int32),
            pltpu.VMEM((2, CHUNK), jnp.int32),
            pltpu.VMEM((CHUNK, D), BF),
            pltpu.VMEM((CHUNK, D), BF),
            pltpu.VMEM_SHARED((NA, D), BF),
            pltpu.SemaphoreType.DMA,
            pltpu.SemaphoreType.DMA,
            pltpu.SemaphoreType.DMA,
            pltpu.SemaphoreType.DMA,
            pltpu.SemaphoreType.DMA,
        ],
        compiler_params=pltpu.CompilerParams(use_tc_tiling_on_sc=False),
    )
    def agg(table, mixed_g, zeros, out,
            ib0, ib1, rb0, rb1, acc, si0, si1, sg0, sg1, semz):
        cid = lax.axis_index("c")
        sid = lax.axis_index("s")
        zcp = pltpu.async_copy(zeros.at[pl.ds(sid * RPT, RPT)],
                               acc.at[pl.ds(sid * RPT, RPT)], semz)
        zcp.wait()
        plsc.subcore_barrier()

        @pl.when(cid == 0)
        def _():
            _pipeline(table, mixed_g, sid * qa, ib0, ib1, rb0, rb1,
                      si0, si1, sg0, sg1, acc, qa)

        @pl.when(cid == 1)
        def _():
            _pipeline(table, mixed_g, NS * qa + sid * qb, ib0, ib1, rb0,
                      rb1, si0, si1, sg0, sg1, acc, qb)

        plsc.subcore_barrier()
        pltpu.sync_copy(acc.at[pl.ds(sid * RPT, RPT)],
                        out.at[pl.ds(cid * N + sid * RPT, RPT)])

    return agg


BM = 2000
GRID = N // BM


def _l1_body(pa, pb, x, wl, wr, b, hl_ref, hr_ref, hb_ref, inv_ref):
    s = pa[...].astype(jnp.float32) + pb[...].astype(jnp.float32)
    cnt = s[:, IN_F:IN_F + 1]
    inv = 1.0 / jnp.maximum(cnt, 1.0)
    agg = s[:, :IN_F] * inv
    h = (jnp.dot(agg, wl[...], preferred_element_type=jnp.float32)
         + jnp.dot(x[...], wr[...], preferred_element_type=jnp.float32)
         + b[...])
    h = jnp.maximum(h, 0.0)
    hl_ref[...] = h[:, :128]
    hr_ref[...] = h[:, 128:]
    hb_ref[...] = h.astype(BF)
    inv_ref[...] = inv


def _tc_l1(parts1, x, Wl1, Wr1, b1r):
    return pl.pallas_call(
        _l1_body,
        grid=(GRID,),
        in_specs=[
            pl.BlockSpec((BM, 160), lambda i: (i, 0)),
            pl.BlockSpec((BM, 160), lambda i: (i + GRID, 0)),
            pl.BlockSpec((BM, IN_F), lambda i: (i, 0)),
            pl.BlockSpec((IN_F, HID), lambda i: (0, 0)),
            pl.BlockSpec((IN_F, HID), lambda i: (0, 0)),
            pl.BlockSpec((1, HID), lambda i: (0, 0)),
        ],
        out_specs=[
            pl.BlockSpec((BM, 128), lambda i: (i, 0)),
            pl.BlockSpec((BM, 128), lambda i: (i, 0)),
            pl.BlockSpec((BM, HID), lambda i: (i, 0)),
            pl.BlockSpec((BM, 1), lambda i: (i, 0)),
        ],
        out_shape=[
            jax.ShapeDtypeStruct((N, 128), jnp.float32),
            jax.ShapeDtypeStruct((N, 128), jnp.float32),
            jax.ShapeDtypeStruct((N, HID), BF),
            jax.ShapeDtypeStruct((N, 1), jnp.float32),
        ],
    )(parts1, parts1, x, Wl1, Wr1, b1r)


def _l2_body(pa, pb, inv, h1l, h1r,
             w2, wr2a, wr2b, b2, wl3p, wr3p, b3p, z_ref, r_ref):
    iv = inv[...]
    agg = (pa[...].astype(jnp.float32) + pb[...].astype(jnp.float32)) * iv
    h2 = (jnp.dot(agg, w2[...], preferred_element_type=jnp.float32)
          + jnp.dot(h1l[...], wr2a[...], preferred_element_type=jnp.float32)
          + jnp.dot(h1r[...], wr2b[...], preferred_element_type=jnp.float32)
          + b2[...])
    h2 = jnp.maximum(h2, 0.0)
    z_ref[...] = jnp.dot(h2, wl3p[...],
                         preferred_element_type=jnp.float32).astype(BF)
    r_ref[...] = (jnp.dot(h2, wr3p[...], preferred_element_type=jnp.float32)
                  + b3p[...])


def _tc_l2(p2, inv, h1l, h1r, w2, wr2a, wr2b, b2r, wl3p, wr3p, b3pr):
    blk = lambda i: (i, 0)
    blk2 = lambda i: (i + GRID, 0)
    full = lambda i: (0, 0)
    return pl.pallas_call(
        _l2_body,
        grid=(GRID,),
        in_specs=[
            pl.BlockSpec((BM, HID), blk),
            pl.BlockSpec((BM, HID), blk2),
            pl.BlockSpec((BM, 1), blk),
            pl.BlockSpec((BM, 128), blk),
            pl.BlockSpec((BM, 128), blk),
            pl.BlockSpec((HID, HID), full),
            pl.BlockSpec((128, HID), full),
            pl.BlockSpec((128, HID), full),
            pl.BlockSpec((1, HID), full),
            pl.BlockSpec((HID, 128), full),
            pl.BlockSpec((HID, 128), full),
            pl.BlockSpec((1, 128), full),
        ],
        out_specs=[
            pl.BlockSpec((BM, 128), blk),
            pl.BlockSpec((BM, 128), blk),
        ],
        out_shape=[
            jax.ShapeDtypeStruct((N, 128), BF),
            jax.ShapeDtypeStruct((N, 128), jnp.float32),
        ],
    )(p2, p2, inv, h1l, h1r, w2, wr2a, wr2b, b2r, wl3p, wr3p, b3pr)


def _l3_body(qa, qb, inv, r, out_ref):
    q = qa[...].astype(jnp.float32) + qb[...].astype(jnp.float32)
    v = q * inv[...] + r[...]
    out_ref[...] = jnp.maximum(v, 0.0)[:, :OUT_F]


def _tc_l3(parts3, inv, r):
    blk = lambda i: (i, 0)
    return pl.pallas_call(
        _l3_body,
        grid=(GRID,),
        in_specs=[
            pl.BlockSpec((BM, 128), blk),
            pl.BlockSpec((BM, 128), lambda i: (i + GRID, 0)),
            pl.BlockSpec((BM, 1), blk),
            pl.BlockSpec((BM, 128), blk),
        ],
        out_specs=pl.BlockSpec((BM, OUT_F), blk),
        out_shape=jax.ShapeDtypeStruct((N, OUT_F), jnp.float32),
    )(parts3, parts3, inv, r)


def kernel(x, edge_index, Wl1, Wr1, b1, Wl2, Wr2, b2, Wl3, Wr3, b3):
    ei = edge_index.astype(jnp.int32)
    src, dst = ei[0], ei[1]

    pad = EP - E
    src_p = jnp.concatenate([src, jnp.zeros((pad,), jnp.int32)])
    dst_p = jnp.concatenate([dst, jnp.full((pad,), N, jnp.int32)])
    mixed_g = jnp.stack([src_p.reshape(TCH, CHUNK),
                         dst_p.reshape(TCH, CHUNK)], axis=1)

    x_pad = jnp.concatenate(
        [x, jnp.ones((N, 1), jnp.float32), jnp.zeros((N, 31), jnp.float32)],
        axis=1).astype(BF)
    z160 = jnp.zeros((N, 160), BF)
    z256 = jnp.zeros((N, HID), BF)
    z128 = jnp.zeros((N, 128), BF)

    b1r = b1.reshape(1, HID)
    wr2a, wr2b = Wr2[:128], Wr2[128:]
    b2r = b2.reshape(1, HID)
    wl3p = jnp.pad(Wl3, ((0, 0), (0, 128 - OUT_F)))
    wr3p = jnp.pad(Wr3, ((0, 0), (0, 128 - OUT_F)))
    b3pr = jnp.pad(b3, (0, 128 - OUT_F)).reshape(1, 128)

    agg160 = _make_agg(160, 114, 44)
    agg256 = _make_agg(HID, 118, 40)
    agg128 = _make_agg(128, 105, 53)

    parts1 = agg160(x_pad, mixed_g, z160)
    h1l, h1r, h1b, inv = _tc_l1(parts1, x, Wl1, Wr1, b1r)

    p2 = agg256(h1b, mixed_g, z256)
    z, r = _tc_l2(p2, inv, h1l, h1r, Wl2, wr2a, wr2b, b2r, wl3p, wr3p, b3pr)

    parts3 = agg128(z, mixed_g, z128)
    return _tc_l3(parts3, inv, r)


# speedup vs baseline: 1.3818x; 1.1993x over previous
import functools

import jax
import jax.numpy as jnp
from jax import lax
from jax.experimental import pallas as pl
from jax.experimental.pallas import tpu as pltpu
from jax.experimental.pallas import tpu_sc as plsc

N = 10000
E = 320000
IN_F = 128
HID = 256
OUT_F = 121

NC = 2
NS = 16
NW = NC * NS
CHUNK = 128
TCH = E // CHUNK
RPT = N // NS
NA = N + 8

BF = jnp.bfloat16


def _mesh():
    return plsc.VectorSubcoreMesh(core_axis_name="c", subcore_axis_name="s",
                                  num_cores=NC, num_subcores=NS)


def _pipeline(table, ei, base, ib0, ib1, rb0, rb1, si0, si1, sg0, sg1,
              acc, nchunk):
    ibufs, rbufs = (ib0, ib1), (rb0, rb1)
    isems, gsems = (si0, si1), (sg0, sg1)

    def iload(c, p):
        off = (base + c) * CHUNK
        pltpu.async_copy(ei.at[0, pl.ds(off, CHUNK)], ibufs[p].at[0],
                         isems[p])
        pltpu.async_copy(ei.at[1, pl.ds(off, CHUNK)], ibufs[p].at[1],
                         isems[p])

    def iwait(c, p):
        off = (base + c) * CHUNK
        pltpu.make_async_copy(ei.at[0, pl.ds(off, CHUNK)],
                              ibufs[p].at[0], isems[p]).wait()
        pltpu.make_async_copy(ei.at[1, pl.ds(off, CHUNK)],
                              ibufs[p].at[1], isems[p]).wait()

    def gstart(p):
        pltpu.async_copy(table.at[ibufs[p].at[0]], rbufs[p], gsems[p])

    def gwait(p):
        pltpu.make_async_copy(table.at[ibufs[p].at[0]], rbufs[p],
                              gsems[p]).wait()

    def scat(p):
        pltpu.sync_copy(rbufs[p], acc.at[ibufs[p].at[1]], add=True)

    iload(0, 0)
    iwait(0, 0)
    gstart(0)
    iload(1, 1)

    def body(c, carry):
        def stage(p):
            iwait(c, p)
            gstart(p)
            gwait(1 - p)
            scat(1 - p)

            @pl.when(c < nchunk - 1)
            def _():
                iload(c + 1, 1 - p)

        @pl.when(c % 2 == 1)
        def _():
            stage(1)

        @pl.when(c % 2 == 0)
        def _():
            stage(0)

        return carry

    lax.fori_loop(1, nchunk, body, 0)
    p = (nchunk - 1) % 2
    gwait(p)
    scat(p)


@functools.lru_cache(maxsize=None)
def _make_agg(D, qa, qb):
    @functools.partial(
        pl.kernel,
        out_type=jax.ShapeDtypeStruct((2 * N, D), BF),
        mesh=_mesh(),
        scratch_types=[
            pltpu.VMEM((2, CHUNK), jnp.int32),
            pltpu.VMEM((2, CHUNK), jnp.int32),
            pltpu.VMEM((CHUNK, D), BF),
            pltpu.VMEM((CHUNK, D), BF),
            pltpu.VMEM_SHARED((NA, D), BF),
            pltpu.SemaphoreType.DMA,
            pltpu.SemaphoreType.DMA,
            pltpu.SemaphoreType.DMA,
            pltpu.SemaphoreType.DMA,
            pltpu.SemaphoreType.DMA,
        ],
        compiler_params=pltpu.CompilerParams(use_tc_tiling_on_sc=False),
    )
    def agg(table, ei, zeros, out,
            ib0, ib1, rb0, rb1, acc, si0, si1, sg0, sg1, semz):
        cid = lax.axis_index("c")
        sid = lax.axis_index("s")
        zcp = pltpu.async_copy(zeros.at[pl.ds(sid * RPT, RPT)],
                               acc.at[pl.ds(sid * RPT, RPT)], semz)
        zcp.wait()
        plsc.subcore_barrier()

        @pl.when((cid == 0) & (sid < 4))
        def _():
            _pipeline(table, ei, sid * (qa + 1), ib0, ib1, rb0, rb1,
                      si0, si1, sg0, sg1, acc, qa + 1)

        @pl.when((cid == 0) & (sid >= 4))
        def _():
            _pipeline(table, ei, 4 + sid * qa, ib0, ib1, rb0, rb1,
                      si0, si1, sg0, sg1, acc, qa)

        @pl.when(cid == 1)
        def _():
            _pipeline(table, ei, NS * qa + 4 + sid * qb, ib0, ib1, rb0,
                      rb1, si0, si1, sg0, sg1, acc, qb)

        plsc.subcore_barrier()
        pltpu.sync_copy(acc.at[pl.ds(sid * RPT, RPT)],
                        out.at[pl.ds(cid * N + sid * RPT, RPT)])

    return agg


BM = 2000
GRID = N // BM


def _l1_body(pa, pb, x, wl, wr, b, hl_ref, hr_ref, hb_ref, inv_ref):
    s = pa[...].astype(jnp.float32) + pb[...].astype(jnp.float32)
    cnt = s[:, IN_F:IN_F + 1]
    inv = 1.0 / jnp.maximum(cnt, 1.0)
    agg = s[:, :IN_F] * inv
    h = (jnp.dot(agg, wl[...], preferred_element_type=jnp.float32)
         + jnp.dot(x[...], wr[...], preferred_element_type=jnp.float32)
         + b[...])
    h = jnp.maximum(h, 0.0)
    hl_ref[...] = h[:, :128]
    hr_ref[...] = h[:, 128:]
    hb_ref[...] = h.astype(BF)
    inv_ref[...] = inv


def _tc_l1(parts1, x, Wl1, Wr1, b1r):
    return pl.pallas_call(
        _l1_body,
        grid=(GRID,),
        in_specs=[
            pl.BlockSpec((BM, 160), lambda i: (i, 0)),
            pl.BlockSpec((BM, 160), lambda i: (i + GRID, 0)),
            pl.BlockSpec((BM, IN_F), lambda i: (i, 0)),
            pl.BlockSpec((IN_F, HID), lambda i: (0, 0)),
            pl.BlockSpec((IN_F, HID), lambda i: (0, 0)),
            pl.BlockSpec((1, HID), lambda i: (0, 0)),
        ],
        out_specs=[
            pl.BlockSpec((BM, 128), lambda i: (i, 0)),
            pl.BlockSpec((BM, 128), lambda i: (i, 0)),
            pl.BlockSpec((BM, HID), lambda i: (i, 0)),
            pl.BlockSpec((BM, 1), lambda i: (i, 0)),
        ],
        out_shape=[
            jax.ShapeDtypeStruct((N, 128), jnp.float32),
            jax.ShapeDtypeStruct((N, 128), jnp.float32),
            jax.ShapeDtypeStruct((N, HID), BF),
            jax.ShapeDtypeStruct((N, 1), jnp.float32),
        ],
    )(parts1, parts1, x, Wl1, Wr1, b1r)


def _l2_body(pa, pb, inv, h1l, h1r,
             w2, wr2a, wr2b, b2, wl3p, wr3p, b3p, z_ref, r_ref):
    iv = inv[...]
    agg = (pa[...].astype(jnp.float32) + pb[...].astype(jnp.float32)) * iv
    h2 = (jnp.dot(agg, w2[...], preferred_element_type=jnp.float32)
          + jnp.dot(h1l[...], wr2a[...], preferred_element_type=jnp.float32)
          + jnp.dot(h1r[...], wr2b[...], preferred_element_type=jnp.float32)
          + b2[...])
    h2 = jnp.maximum(h2, 0.0)
    z_ref[...] = jnp.dot(h2, wl3p[...],
                         preferred_element_type=jnp.float32).astype(BF)
    r_ref[...] = (jnp.dot(h2, wr3p[...], preferred_element_type=jnp.float32)
                  + b3p[...])


def _tc_l2(p2, inv, h1l, h1r, w2, wr2a, wr2b, b2r, wl3p, wr3p, b3pr):
    blk = lambda i: (i, 0)
    blk2 = lambda i: (i + GRID, 0)
    full = lambda i: (0, 0)
    return pl.pallas_call(
        _l2_body,
        grid=(GRID,),
        in_specs=[
            pl.BlockSpec((BM, HID), blk),
            pl.BlockSpec((BM, HID), blk2),
            pl.BlockSpec((BM, 1), blk),
            pl.BlockSpec((BM, 128), blk),
            pl.BlockSpec((BM, 128), blk),
            pl.BlockSpec((HID, HID), full),
            pl.BlockSpec((128, HID), full),
            pl.BlockSpec((128, HID), full),
            pl.BlockSpec((1, HID), full),
            pl.BlockSpec((HID, 128), full),
            pl.BlockSpec((HID, 128), full),
            pl.BlockSpec((1, 128), full),
        ],
        out_specs=[
            pl.BlockSpec((BM, 128), blk),
            pl.BlockSpec((BM, 128), blk),
        ],
        out_shape=[
            jax.ShapeDtypeStruct((N, 128), BF),
            jax.ShapeDtypeStruct((N, 128), jnp.float32),
        ],
    )(p2, p2, inv, h1l, h1r, w2, wr2a, wr2b, b2r, wl3p, wr3p, b3pr)


def _l3_body(qa, qb, inv, r, out_ref):
    q = qa[...].astype(jnp.float32) + qb[...].astype(jnp.float32)
    v = q * inv[...] + r[...]
    out_ref[...] = jnp.maximum(v, 0.0)[:, :OUT_F]


def _tc_l3(parts3, inv, r):
    blk = lambda i: (i, 0)
    return pl.pallas_call(
        _l3_body,
        grid=(GRID,),
        in_specs=[
            pl.BlockSpec((BM, 128), blk),
            pl.BlockSpec((BM, 128), lambda i: (i + GRID, 0)),
            pl.BlockSpec((BM, 1), blk),
            pl.BlockSpec((BM, 128), blk),
        ],
        out_specs=pl.BlockSpec((BM, OUT_F), blk),
        out_shape=jax.ShapeDtypeStruct((N, OUT_F), jnp.float32),
    )(parts3, parts3, inv, r)


def kernel(x, edge_index, Wl1, Wr1, b1, Wl2, Wr2, b2, Wl3, Wr3, b3):
    ei = edge_index.astype(jnp.int32)

    x_pad = jnp.concatenate(
        [x, jnp.ones((N, 1), jnp.float32), jnp.zeros((N, 31), jnp.float32)],
        axis=1).astype(BF)
    z160 = jnp.zeros((N, 160), BF)
    z256 = jnp.zeros((N, HID), BF)
    z128 = jnp.zeros((N, 128), BF)

    b1r = b1.reshape(1, HID)
    wr2a, wr2b = Wr2[:128], Wr2[128:]
    b2r = b2.reshape(1, HID)
    wl3p = jnp.pad(Wl3, ((0, 0), (0, 128 - OUT_F)))
    wr3p = jnp.pad(Wr3, ((0, 0), (0, 128 - OUT_F)))
    b3pr = jnp.pad(b3, (0, 128 - OUT_F)).reshape(1, 128)

    agg160 = _make_agg(160, 117, 39)
    agg256 = _make_agg(HID, 127, 29)
    agg128 = _make_agg(128, 105, 51)

    parts1 = agg160(x_pad, ei, z160)
    h1l, h1r, h1b, inv = _tc_l1(parts1, x, Wl1, Wr1, b1r)

    p2 = agg256(h1b, ei, z256)
    z, r = _tc_l2(p2, inv, h1l, h1r, Wl2, wr2a, wr2b, b2r, wl3p, wr3p, b3pr)

    parts3 = agg128(z, ei, z128)
    return _tc_l3(parts3, inv, r)


# speedup vs baseline: 1.6489x; 1.1933x over previous
import functools

import jax
import jax.numpy as jnp
from jax import lax
from jax.experimental import pallas as pl
from jax.experimental.pallas import tpu as pltpu
from jax.experimental.pallas import tpu_sc as plsc

N = 10000
E = 320000
IN_F = 128
HID = 256
OUT_F = 121

NC = 2
NS = 16
NW = NC * NS
CHUNK = 128
TCH = E // CHUNK
RPT = N // NS
NA = N + 8

BF = jnp.bfloat16


def _mesh():
    return plsc.VectorSubcoreMesh(core_axis_name="c", subcore_axis_name="s",
                                  num_cores=NC, num_subcores=NS)


def _pipeline(table, ei, base, ib0, ib1, rb0, rb1, si0, si1, sg0, sg1,
              acc, nchunk):
    ibufs, rbufs = (ib0, ib1), (rb0, rb1)
    isems, gsems = (si0, si1), (sg0, sg1)

    def iload(c, p):
        off = (base + c) * CHUNK
        pltpu.async_copy(ei.at[0, pl.ds(off, CHUNK)], ibufs[p].at[0],
                         isems[p])
        pltpu.async_copy(ei.at[1, pl.ds(off, CHUNK)], ibufs[p].at[1],
                         isems[p])

    def iwait(c, p):
        off = (base + c) * CHUNK
        pltpu.make_async_copy(ei.at[0, pl.ds(off, CHUNK)],
                              ibufs[p].at[0], isems[p]).wait()
        pltpu.make_async_copy(ei.at[1, pl.ds(off, CHUNK)],
                              ibufs[p].at[1], isems[p]).wait()

    def gstart(p):
        pltpu.async_copy(table.at[ibufs[p].at[0]], rbufs[p], gsems[p])

    def gwait(p):
        pltpu.make_async_copy(table.at[ibufs[p].at[0]], rbufs[p],
                              gsems[p]).wait()

    def scat(p):
        pltpu.sync_copy(rbufs[p], acc.at[ibufs[p].at[1]], add=True)

    iload(0, 0)
    iwait(0, 0)
    gstart(0)
    iload(1, 1)

    def body(c, carry):
        def stage(p):
            iwait(c, p)
            gstart(p)
            gwait(1 - p)
            scat(1 - p)

            @pl.when(c < nchunk - 1)
            def _():
                iload(c + 1, 1 - p)

        @pl.when(c % 2 == 1)
        def _():
            stage(1)

        @pl.when(c % 2 == 0)
        def _():
            stage(0)

        return carry

    lax.fori_loop(1, nchunk, body, 0)
    p = (nchunk - 1) % 2
    gwait(p)
    scat(p)


@functools.lru_cache(maxsize=None)
def _make_agg(D, qa, qb):
    @functools.partial(
        pl.kernel,
        out_type=jax.ShapeDtypeStruct((2 * N, D), BF),
        mesh=_mesh(),
        scratch_types=[
            pltpu.VMEM((2, CHUNK), jnp.int32),
            pltpu.VMEM((2, CHUNK), jnp.int32),
            pltpu.VMEM((CHUNK, D), BF),
            pltpu.VMEM((CHUNK, D), BF),
            pltpu.VMEM_SHARED((NA, D), BF),
            pltpu.SemaphoreType.DMA,
            pltpu.SemaphoreType.DMA,
            pltpu.SemaphoreType.DMA,
            pltpu.SemaphoreType.DMA,
            pltpu.SemaphoreType.DMA,
        ],
        compiler_params=pltpu.CompilerParams(use_tc_tiling_on_sc=False),
    )
    def agg(table, ei, zeros, out,
            ib0, ib1, rb0, rb1, acc, si0, si1, sg0, sg1, semz):
        cid = lax.axis_index("c")
        sid = lax.axis_index("s")
        zcp = pltpu.async_copy(zeros.at[pl.ds(sid * RPT, RPT)],
                               acc.at[pl.ds(sid * RPT, RPT)], semz)
        zcp.wait()
        plsc.subcore_barrier()

        @pl.when((cid == 0) & (sid < 4))
        def _():
            _pipeline(table, ei, sid * (qa + 1), ib0, ib1, rb0, rb1,
                      si0, si1, sg0, sg1, acc, qa + 1)

        @pl.when((cid == 0) & (sid >= 4))
        def _():
            _pipeline(table, ei, 4 + sid * qa, ib0, ib1, rb0, rb1,
                      si0, si1, sg0, sg1, acc, qa)

        @pl.when(cid == 1)
        def _():
            _pipeline(table, ei, NS * qa + 4 + sid * qb, ib0, ib1, rb0,
                      rb1, si0, si1, sg0, sg1, acc, qb)

        plsc.subcore_barrier()
        pltpu.sync_copy(acc.at[pl.ds(sid * RPT, RPT)],
                        out.at[pl.ds(cid * N + sid * RPT, RPT)])

    return agg


BM = 2000
GRID = N // BM


def _l1_body(pa, pb, x, wl, wr, b, hl_ref, hr_ref, hb_ref, inv_ref):
    s = pa[...].astype(jnp.float32) + pb[...].astype(jnp.float32)
    cnt = s[:, IN_F:IN_F + 1]
    inv = 1.0 / jnp.maximum(cnt, 1.0)
    agg = s[:, :IN_F] * inv
    h = (jnp.dot(agg, wl[...], preferred_element_type=jnp.float32)
         + jnp.dot(x[...], wr[...], preferred_element_type=jnp.float32)
         + b[...])
    h = jnp.maximum(h, 0.0)
    hl_ref[...] = h[:, :128]
    hr_ref[...] = h[:, 128:]
    hb_ref[...] = h.astype(BF)
    inv_ref[...] = inv


def _tc_l1(parts1, x, Wl1, Wr1, b1r):
    return pl.pallas_call(
        _l1_body,
        grid=(GRID,),
        in_specs=[
            pl.BlockSpec((BM, 160), lambda i: (i, 0)),
            pl.BlockSpec((BM, 160), lambda i: (i + GRID, 0)),
            pl.BlockSpec((BM, IN_F), lambda i: (i, 0)),
            pl.BlockSpec((IN_F, HID), lambda i: (0, 0)),
            pl.BlockSpec((IN_F, HID), lambda i: (0, 0)),
            pl.BlockSpec((1, HID), lambda i: (0, 0)),
        ],
        out_specs=[
            pl.BlockSpec((BM, 128), lambda i: (i, 0)),
            pl.BlockSpec((BM, 128), lambda i: (i, 0)),
            pl.BlockSpec((BM, HID), lambda i: (i, 0)),
            pl.BlockSpec((BM, 1), lambda i: (i, 0)),
        ],
        out_shape=[
            jax.ShapeDtypeStruct((N, 128), jnp.float32),
            jax.ShapeDtypeStruct((N, 128), jnp.float32),
            jax.ShapeDtypeStruct((N, HID), BF),
            jax.ShapeDtypeStruct((N, 1), jnp.float32),
        ],
    )(parts1, parts1, x, Wl1, Wr1, b1r)


def _l2_body(pa, pb, inv, h1l, h1r,
             w2, wr2a, wr2b, b2, wl3p, wr3p, b3p, z_ref, r_ref):
    iv = inv[...]
    agg = (pa[...].astype(jnp.float32) + pb[...].astype(jnp.float32)) * iv
    h2 = (jnp.dot(agg, w2[...], preferred_element_type=jnp.float32)
          + jnp.dot(h1l[...], wr2a[...], preferred_element_type=jnp.float32)
          + jnp.dot(h1r[...], wr2b[...], preferred_element_type=jnp.float32)
          + b2[...])
    h2 = jnp.maximum(h2, 0.0)
    z_ref[...] = jnp.dot(h2, wl3p[...],
                         preferred_element_type=jnp.float32).astype(BF)
    r_ref[...] = (jnp.dot(h2, wr3p[...], preferred_element_type=jnp.float32)
                  + b3p[...])


def _tc_l2(p2, inv, h1l, h1r, w2, wr2a, wr2b, b2r, wl3p, wr3p, b3pr):
    blk = lambda i: (i, 0)
    blk2 = lambda i: (i + GRID, 0)
    full = lambda i: (0, 0)
    return pl.pallas_call(
        _l2_body,
        grid=(GRID,),
        in_specs=[
            pl.BlockSpec((BM, HID), blk),
            pl.BlockSpec((BM, HID), blk2),
            pl.BlockSpec((BM, 1), blk),
            pl.BlockSpec((BM, 128), blk),
            pl.BlockSpec((BM, 128), blk),
            pl.BlockSpec((HID, HID), full),
            pl.BlockSpec((128, HID), full),
            pl.BlockSpec((128, HID), full),
            pl.BlockSpec((1, HID), full),
            pl.BlockSpec((HID, 128), full),
            pl.BlockSpec((HID, 128), full),
            pl.BlockSpec((1, 128), full),
        ],
        out_specs=[
            pl.BlockSpec((BM, 128), blk),
            pl.BlockSpec((BM, 128), blk),
        ],
        out_shape=[
            jax.ShapeDtypeStruct((N, 128), BF),
            jax.ShapeDtypeStruct((N, 128), jnp.float32),
        ],
    )(p2, p2, inv, h1l, h1r, w2, wr2a, wr2b, b2r, wl3p, wr3p, b3pr)


def _l3_body(qa, qb, inv, r, out_ref):
    q = qa[...].astype(jnp.float32) + qb[...].astype(jnp.float32)
    v = q * inv[...] + r[...]
    out_ref[...] = jnp.maximum(v, 0.0)[:, :OUT_F]


def _tc_l3(parts3, inv, r):
    blk = lambda i: (i, 0)
    return pl.pallas_call(
        _l3_body,
        grid=(GRID,),
        in_specs=[
            pl.BlockSpec((BM, 128), blk),
            pl.BlockSpec((BM, 128), lambda i: (i + GRID, 0)),
            pl.BlockSpec((BM, 1), blk),
            pl.BlockSpec((BM, 128), blk),
        ],
        out_specs=pl.BlockSpec((BM, OUT_F), blk),
        out_shape=jax.ShapeDtypeStruct((N, OUT_F), jnp.float32),
    )(parts3, parts3, inv, r)


def kernel(x, edge_index, Wl1, Wr1, b1, Wl2, Wr2, b2, Wl3, Wr3, b3):
    ei = edge_index.astype(jnp.int32)

    x_pad = jnp.concatenate(
        [x, jnp.ones((N, 1), jnp.float32), jnp.zeros((N, 31), jnp.float32)],
        axis=1).astype(BF)
    z160 = jnp.zeros((N, 160), BF)
    z256 = jnp.zeros((N, HID), BF)
    z128 = jnp.zeros((N, 128), BF)

    b1r = b1.reshape(1, HID)
    wr2a, wr2b = Wr2[:128], Wr2[128:]
    b2r = b2.reshape(1, HID)
    wl3p = jnp.pad(Wl3, ((0, 0), (0, 128 - OUT_F)))
    wr3p = jnp.pad(Wr3, ((0, 0), (0, 128 - OUT_F)))
    b3pr = jnp.pad(b3, (0, 128 - OUT_F)).reshape(1, 128)

    agg160 = _make_agg(160, 84, 72)
    agg256 = _make_agg(HID, 88, 68)
    agg128 = _make_agg(128, 81, 75)

    parts1 = agg160(x_pad, ei, z160)
    h1l, h1r, h1b, inv = _tc_l1(parts1, x, Wl1, Wr1, b1r)

    p2 = agg256(h1b, ei, z256)
    z, r = _tc_l2(p2, inv, h1l, h1r, Wl2, wr2a, wr2b, b2r, wl3p, wr3p, b3pr)

    parts3 = agg128(z, ei, z128)
    return _tc_l3(parts3, inv, r)


# speedup vs baseline: 1.7186x; 1.0423x over previous
import functools

import jax
import jax.numpy as jnp
from jax import lax
from jax.experimental import pallas as pl
from jax.experimental.pallas import tpu as pltpu
from jax.experimental.pallas import tpu_sc as plsc

N = 10000
E = 320000
IN_F = 128
HID = 256
OUT_F = 121

NC = 2
NS = 16
NW = NC * NS
CHUNK = 128
TCH = E // CHUNK
RPT = N // NS
NA = N + 8

BF = jnp.bfloat16


def _mesh():
    return plsc.VectorSubcoreMesh(core_axis_name="c", subcore_axis_name="s",
                                  num_cores=NC, num_subcores=NS)


def _pipeline(table, ei, base, ib0, ib1, rb0, rb1, si0, si1, sg0, sg1,
              acc, nchunk):
    ibufs, rbufs = (ib0, ib1), (rb0, rb1)
    isems, gsems = (si0, si1), (sg0, sg1)

    def iload(c, p):
        off = (base + c) * CHUNK
        pltpu.async_copy(ei.at[0, pl.ds(off, CHUNK)], ibufs[p].at[0],
                         isems[p])
        pltpu.async_copy(ei.at[1, pl.ds(off, CHUNK)], ibufs[p].at[1],
                         isems[p])

    def iwait(c, p):
        off = (base + c) * CHUNK
        pltpu.make_async_copy(ei.at[0, pl.ds(off, CHUNK)],
                              ibufs[p].at[0], isems[p]).wait()
        pltpu.make_async_copy(ei.at[1, pl.ds(off, CHUNK)],
                              ibufs[p].at[1], isems[p]).wait()

    def gstart(p):
        pltpu.async_copy(table.at[ibufs[p].at[0]], rbufs[p], gsems[p])

    def gwait(p):
        pltpu.make_async_copy(table.at[ibufs[p].at[0]], rbufs[p],
                              gsems[p]).wait()

    def scat(p):
        pltpu.sync_copy(rbufs[p], acc.at[ibufs[p].at[1]], add=True)

    iload(0, 0)
    iwait(0, 0)
    gstart(0)
    iload(1, 1)

    def body(c, carry):
        def stage(p):
            iwait(c, p)
            gstart(p)
            gwait(1 - p)
            scat(1 - p)

            @pl.when(c < nchunk - 1)
            def _():
                iload(c + 1, 1 - p)

        @pl.when(c % 2 == 1)
        def _():
            stage(1)

        @pl.when(c % 2 == 0)
        def _():
            stage(0)

        return carry

    lax.fori_loop(1, nchunk, body, 0)
    p = (nchunk - 1) % 2
    gwait(p)
    scat(p)


@functools.lru_cache(maxsize=None)
def _make_agg(D, qa, qb):
    @functools.partial(
        pl.kernel,
        out_type=jax.ShapeDtypeStruct((2 * N, D), BF),
        mesh=_mesh(),
        scratch_types=[
            pltpu.VMEM((2, CHUNK), jnp.int32),
            pltpu.VMEM((2, CHUNK), jnp.int32),
            pltpu.VMEM((CHUNK, D), BF),
            pltpu.VMEM((CHUNK, D), BF),
            pltpu.VMEM_SHARED((NA, D), BF),
            pltpu.SemaphoreType.DMA,
            pltpu.SemaphoreType.DMA,
            pltpu.SemaphoreType.DMA,
            pltpu.SemaphoreType.DMA,
            pltpu.SemaphoreType.DMA,
        ],
        compiler_params=pltpu.CompilerParams(use_tc_tiling_on_sc=False),
    )
    def agg(table, ei, zeros, out,
            ib0, ib1, rb0, rb1, acc, si0, si1, sg0, sg1, semz):
        cid = lax.axis_index("c")
        sid = lax.axis_index("s")
        zcp = pltpu.async_copy(zeros.at[pl.ds(sid * RPT, RPT)],
                               acc.at[pl.ds(sid * RPT, RPT)], semz)
        zcp.wait()
        plsc.subcore_barrier()

        @pl.when((cid == 0) & (sid < 4))
        def _():
            _pipeline(table, ei, sid * (qa + 1), ib0, ib1, rb0, rb1,
                      si0, si1, sg0, sg1, acc, qa + 1)

        @pl.when((cid == 0) & (sid >= 4))
        def _():
            _pipeline(table, ei, 4 + sid * qa, ib0, ib1, rb0, rb1,
                      si0, si1, sg0, sg1, acc, qa)

        @pl.when(cid == 1)
        def _():
            _pipeline(table, ei, NS * qa + 4 + sid * qb, ib0, ib1, rb0,
                      rb1, si0, si1, sg0, sg1, acc, qb)

        plsc.subcore_barrier()
        pltpu.sync_copy(acc.at[pl.ds(sid * RPT, RPT)],
                        out.at[pl.ds(cid * N + sid * RPT, RPT)])

    return agg


BM = 2000
GRID = N // BM


def _l1_body(pa, pb, x, wl, wr, b, hl_ref, hr_ref, hb_ref, inv_ref):
    s = pa[...].astype(jnp.float32) + pb[...].astype(jnp.float32)
    cnt = s[:, IN_F:IN_F + 1]
    inv = 1.0 / jnp.maximum(cnt, 1.0)
    agg = s[:, :IN_F] * inv
    h = (jnp.dot(agg, wl[...], preferred_element_type=jnp.float32)
         + jnp.dot(x[...], wr[...], preferred_element_type=jnp.float32)
         + b[...])
    h = jnp.maximum(h, 0.0)
    hl_ref[...] = h[:, :128]
    hr_ref[...] = h[:, 128:]
    hb_ref[...] = h.astype(BF)
    inv_ref[...] = inv


def _tc_l1(parts1, x, Wl1, Wr1, b1r):
    return pl.pallas_call(
        _l1_body,
        grid=(GRID,),
        in_specs=[
            pl.BlockSpec((BM, 160), lambda i: (i, 0)),
            pl.BlockSpec((BM, 160), lambda i: (i + GRID, 0)),
            pl.BlockSpec((BM, IN_F), lambda i: (i, 0)),
            pl.BlockSpec((IN_F, HID), lambda i: (0, 0)),
            pl.BlockSpec((IN_F, HID), lambda i: (0, 0)),
            pl.BlockSpec((1, HID), lambda i: (0, 0)),
        ],
        out_specs=[
            pl.BlockSpec((BM, 128), lambda i: (i, 0)),
            pl.BlockSpec((BM, 128), lambda i: (i, 0)),
            pl.BlockSpec((BM, HID), lambda i: (i, 0)),
            pl.BlockSpec((BM, 1), lambda i: (i, 0)),
        ],
        out_shape=[
            jax.ShapeDtypeStruct((N, 128), jnp.float32),
            jax.ShapeDtypeStruct((N, 128), jnp.float32),
            jax.ShapeDtypeStruct((N, HID), BF),
            jax.ShapeDtypeStruct((N, 1), jnp.float32),
        ],
    )(parts1, parts1, x, Wl1, Wr1, b1r)


def _l2_body(pa, pb, inv, h1l, h1r,
             w2, wr2a, wr2b, b2, wl3p, wr3p, b3p, z_ref, r_ref):
    iv = inv[...]
    agg = (pa[...].astype(jnp.float32) + pb[...].astype(jnp.float32)) * iv
    h2 = (jnp.dot(agg, w2[...], preferred_element_type=jnp.float32)
          + jnp.dot(h1l[...], wr2a[...], preferred_element_type=jnp.float32)
          + jnp.dot(h1r[...], wr2b[...], preferred_element_type=jnp.float32)
          + b2[...])
    h2 = jnp.maximum(h2, 0.0)
    z_ref[...] = jnp.dot(h2, wl3p[...],
                         preferred_element_type=jnp.float32).astype(BF)
    r_ref[...] = (jnp.dot(h2, wr3p[...], preferred_element_type=jnp.float32)
                  + b3p[...])


def _tc_l2(p2, inv, h1l, h1r, w2, wr2a, wr2b, b2r, wl3p, wr3p, b3pr):
    blk = lambda i: (i, 0)
    blk2 = lambda i: (i + GRID, 0)
    full = lambda i: (0, 0)
    return pl.pallas_call(
        _l2_body,
        grid=(GRID,),
        in_specs=[
            pl.BlockSpec((BM, HID), blk),
            pl.BlockSpec((BM, HID), blk2),
            pl.BlockSpec((BM, 1), blk),
            pl.BlockSpec((BM, 128), blk),
            pl.BlockSpec((BM, 128), blk),
            pl.BlockSpec((HID, HID), full),
            pl.BlockSpec((128, HID), full),
            pl.BlockSpec((128, HID), full),
            pl.BlockSpec((1, HID), full),
            pl.BlockSpec((HID, 128), full),
            pl.BlockSpec((HID, 128), full),
            pl.BlockSpec((1, 128), full),
        ],
        out_specs=[
            pl.BlockSpec((BM, 128), blk),
            pl.BlockSpec((BM, 128), blk),
        ],
        out_shape=[
            jax.ShapeDtypeStruct((N, 128), BF),
            jax.ShapeDtypeStruct((N, 128), jnp.float32),
        ],
    )(p2, p2, inv, h1l, h1r, w2, wr2a, wr2b, b2r, wl3p, wr3p, b3pr)


def _l3_body(qa, qb, inv, r, out_ref):
    q = qa[...].astype(jnp.float32) + qb[...].astype(jnp.float32)
    v = q * inv[...] + r[...]
    out_ref[...] = jnp.maximum(v, 0.0)[:, :OUT_F]


def _tc_l3(parts3, inv, r):
    blk = lambda i: (i, 0)
    return pl.pallas_call(
        _l3_body,
        grid=(GRID,),
        in_specs=[
            pl.BlockSpec((BM, 128), blk),
            pl.BlockSpec((BM, 128), lambda i: (i + GRID, 0)),
            pl.BlockSpec((BM, 1), blk),
            pl.BlockSpec((BM, 128), blk),
        ],
        out_specs=pl.BlockSpec((BM, OUT_F), blk),
        out_shape=jax.ShapeDtypeStruct((N, OUT_F), jnp.float32),
    )(parts3, parts3, inv, r)


def kernel(x, edge_index, Wl1, Wr1, b1, Wl2, Wr2, b2, Wl3, Wr3, b3):
    ei = edge_index.astype(jnp.int32)

    x_pad = jnp.concatenate(
        [x, jnp.ones((N, 1), jnp.float32), jnp.zeros((N, 31), jnp.float32)],
        axis=1).astype(BF)
    z160 = jnp.zeros((N, 160), BF)
    z256 = jnp.zeros((N, HID), BF)
    z128 = jnp.zeros((N, 128), BF)

    b1r = b1.reshape(1, HID)
    wr2a, wr2b = Wr2[:128], Wr2[128:]
    b2r = b2.reshape(1, HID)
    wl3p = jnp.pad(Wl3, ((0, 0), (0, 128 - OUT_F)))
    wr3p = jnp.pad(Wr3, ((0, 0), (0, 128 - OUT_F)))
    b3pr = jnp.pad(b3, (0, 128 - OUT_F)).reshape(1, 128)

    agg160 = _make_agg(160, 78, 78)
    agg256 = _make_agg(HID, 78, 78)
    agg128 = _make_agg(128, 78, 78)

    parts1 = agg160(x_pad, ei, z160)
    h1l, h1r, h1b, inv = _tc_l1(parts1, x, Wl1, Wr1, b1r)

    p2 = agg256(h1b, ei, z256)
    z, r = _tc_l2(p2, inv, h1l, h1r, Wl2, wr2a, wr2b, b2r, wl3p, wr3p, b3pr)

    parts3 = agg128(z, ei, z128)
    return _tc_l3(parts3, inv, r)
